# baseline (speedup 1.0000x reference)
import functools
import jax
import jax.numpy as jnp
from jax import lax
from jax.experimental import pallas as pl
from jax.experimental.pallas import tpu as pltpu
from jax.experimental.pallas import tpu_sc as plsc

_B, _T, _D = 4, 2048, 4096
_NC, _NS = 2, 16
_NW = _NC * _NS
_ROWS_W = _T // _NW
_CS = 4
_CHUNK = _CS * _D
_NCH = _ROWS_W // _CS
_L = 16

_mesh = plsc.VectorSubcoreMesh(core_axis_name="c", subcore_axis_name="s")


@functools.partial(
    pl.kernel,
    out_type=jax.ShapeDtypeStruct((_B * _T * _D,), jnp.float32),
    mesh=_mesh,
    scratch_types=[
        pltpu.VMEM((_CHUNK,), jnp.float32),
        pltpu.VMEM((_B, _CHUNK), jnp.float32),
    ],
)
def _sc_add(x_hbm, pe_hbm, out_hbm, pe_v, x_v):
    wid = lax.axis_index("s") * _NC + lax.axis_index("c")
    base = wid * _ROWS_W * _D

    def chunk_body(c, carry):
        off = base + c * _CHUNK
        pltpu.sync_copy(pe_hbm.at[pl.ds(off, _CHUNK)], pe_v)
        for b in range(_B):
            pltpu.sync_copy(x_hbm.at[pl.ds(b * _T * _D + off, _CHUNK)],
                            x_v.at[b])

        def add_body(i, carry2):
            sl = pl.ds(i * _L, _L)
            pv = pe_v[sl]
            for b in range(_B):
                x_v[b, sl] = x_v[b, sl] + pv
            return carry2

        lax.fori_loop(0, _CHUNK // _L, add_body, 0, unroll=4)

        for b in range(_B):
            pltpu.sync_copy(x_v.at[b],
                            out_hbm.at[pl.ds(b * _T * _D + off, _CHUNK)])
        return carry

    lax.fori_loop(0, _NCH, chunk_body, 0)


def kernel(x, pe):
    B, T, D = x.shape
    out = _sc_add(x.reshape(-1), pe[:T].reshape(-1))
    return out.reshape(B, T, D)


# speedup vs baseline: 1.4136x; 1.4136x over previous
import functools
import jax
import jax.numpy as jnp
from jax import lax
from jax.experimental import pallas as pl
from jax.experimental.pallas import tpu as pltpu
from jax.experimental.pallas import tpu_sc as plsc

_B, _T, _D = 4, 2048, 4096
_NC, _NS = 2, 16
_NW = _NC * _NS
_ROWS_W = _T // _NW
_CS = 4
_NCH = _ROWS_W // _CS
_L = 16

_mesh = plsc.VectorSubcoreMesh(core_axis_name="c", subcore_axis_name="s")


@functools.partial(
    pl.kernel,
    out_type=jax.ShapeDtypeStruct((_B, _T, _D), jnp.float32),
    mesh=_mesh,
    scratch_types=[
        pltpu.VMEM((_CS, _D), jnp.float32),
        pltpu.VMEM((_B, _CS, _D), jnp.float32),
    ],
)
def _sc_add(x_hbm, pe_hbm, out_hbm, pe_v, x_v):
    wid = lax.axis_index("s") * _NC + lax.axis_index("c")
    base = wid * _ROWS_W

    def chunk_body(c, carry):
        row0 = base + c * _CS
        pltpu.sync_copy(pe_hbm.at[pl.ds(row0, _CS)], pe_v)
        for b in range(_B):
            pltpu.sync_copy(x_hbm.at[b, pl.ds(row0, _CS)], x_v.at[b])

        for r in range(_CS):
            def add_body(i, carry2, r=r):
                sl = pl.ds(i * _L, _L)
                pv = pe_v[r, sl]
                for b in range(_B):
                    x_v[b, r, sl] = x_v[b, r, sl] + pv
                return carry2

            lax.fori_loop(0, _D // _L, add_body, 0, unroll=4)

        for b in range(_B):
            pltpu.sync_copy(x_v.at[b], out_hbm.at[b, pl.ds(row0, _CS)])
        return carry

    lax.fori_loop(0, _NCH, chunk_body, 0)


def kernel(x, pe):
    B, T, D = x.shape
    return _sc_add(x, pe[:T])


# speedup vs baseline: 5.4070x; 3.8250x over previous
import functools
import jax
import jax.numpy as jnp
from jax import lax
from jax.experimental import pallas as pl
from jax.experimental.pallas import tpu as pltpu
from jax.experimental.pallas import tpu_sc as plsc

_B, _T, _D = 4, 2048, 4096
_NC, _NS = 2, 16
_NW = _NC * _NS
_ROWS_W = _T // _NW
_CS = 2
_NCH = _ROWS_W // _CS
_L = 16
_K = 3
_PF = 2

_mesh = plsc.VectorSubcoreMesh(core_axis_name="c", subcore_axis_name="s")


@functools.partial(
    pl.kernel,
    out_type=jax.ShapeDtypeStruct((_B, _T, _D), jnp.float32),
    mesh=_mesh,
    scratch_types=[
        pltpu.VMEM((_K, _CS, _D), jnp.float32),
        pltpu.VMEM((_K, _B, _CS, _D), jnp.float32),
        pltpu.SemaphoreType.DMA,
        pltpu.SemaphoreType.DMA,
        pltpu.SemaphoreType.DMA,
        pltpu.SemaphoreType.DMA,
        pltpu.SemaphoreType.DMA,
        pltpu.SemaphoreType.DMA,
    ],
)
def _sc_add(x_hbm, pe_hbm, out_hbm, pe_v, x_v, si0, si1, si2, so0, so1, so2):
    sin = (si0, si1, si2)
    sout = (so0, so1, so2)
    wid = lax.axis_index("s") * _NC + lax.axis_index("c")
    base = wid * _ROWS_W

    def fire_in(c, k):
        row0 = base + c * _CS
        pltpu.async_copy(pe_hbm.at[pl.ds(row0, _CS)], pe_v.at[k], sin[k])
        for b in range(_B):
            pltpu.async_copy(x_hbm.at[b, pl.ds(row0, _CS)], x_v.at[k, b],
                             sin[k])

    def wait_in(k):
        for _ in range(_B + 1):
            pltpu.make_async_copy(pe_hbm.at[pl.ds(0, _CS)], pe_v.at[k],
                                  sin[k]).wait()

    def fire_out(c, k):
        row0 = base + c * _CS
        for b in range(_B):
            pltpu.async_copy(x_v.at[k, b], out_hbm.at[b, pl.ds(row0, _CS)],
                             sout[k])

    def wait_out(k):
        for _ in range(_B):
            pltpu.make_async_copy(x_v.at[k, 0], out_hbm.at[0, pl.ds(0, _CS)],
                                  sout[k]).wait()

    def compute(k):
        for r in range(_CS):
            @plsc.parallel_loop(0, _D // _L, unroll=8)
            def _(i, r=r, k=k):
                sl = pl.ds(i * _L, _L)
                pv = pe_v[k, r, sl]
                for b in range(_B):
                    x_v[k, b, r, sl] = x_v[k, b, r, sl] + pv

    for c0 in range(_PF):
        fire_in(c0, c0 % _K)

    @pl.loop(0, _NCH)
    def _(c):
        k = lax.rem(c, _K)
        for kk in range(_K):
            @pl.when(k == kk)
            def _(kk=kk):
                wait_in(kk)
                compute(kk)
                fire_out(c, kk)
        kp = lax.rem(c + _PF, _K)

        @pl.when(c + _PF < _NCH)
        def _():
            for kk in range(_K):
                @pl.when(kp == kk)
                def _(kk=kk):
                    @pl.when(c + _PF >= _K)
                    def _():
                        wait_out(kk)
                    fire_in(c + _PF, kk)

    for kk in range(_K):
        wait_out(kk)


def kernel(x, pe):
    B, T, D = x.shape
    return _sc_add(x, pe[:T])
